# Initial kernel scaffold; baseline (speedup 1.0000x reference)
import jax
import jax.numpy as jnp
from jax.experimental import pallas as pl


def kernel(inputs, targets):
    raise NotImplementedError("write your pallas kernel here")


# speedup vs baseline: 15.4896x; 15.4896x over previous
import functools

import jax
import jax.numpy as jnp
from jax import lax
from jax.experimental import pallas as pl
from jax.experimental.pallas import tpu as pltpu

_ALPHA = 40.0
_BETA = 10.0
_K = 64
_MAX_FINITE_BITS = 0x7F7FFFFF


def _nca_body(x_ref, xt_ref, tcol_ref, trow_ref, loss_ref, posd_ref, negd_ref):
    x = x_ref[...]
    xt = xt_ref[...]
    tcol = tcol_ref[...]
    trow = trow_ref[...]
    n = x.shape[0]

    g = lax.dot_general(x, xt, (((1,), (0,)), ((), ())),
                        preferred_element_type=jnp.float32)
    x2_col = jnp.sum(x * x, axis=1, keepdims=True)
    x2_row = jnp.sum(xt * xt, axis=0, keepdims=True)
    d2 = x2_col + x2_row - 2.0 * g
    dist = jnp.sqrt(jnp.maximum(d2, 1e-12))

    r = lax.broadcasted_iota(jnp.int32, (n, n), 0)
    c = lax.broadcasted_iota(jnp.int32, (n, n), 1)
    eye = r == c
    same = tcol == trow
    pos_mask = same & (~eye)
    neg_mask = ~same

    inf = jnp.float32(jnp.inf)
    dsel = jnp.where(eye, inf, dist)

    lo = jnp.zeros((n, 1), jnp.int32)
    hi = jnp.full((n, 1), _MAX_FINITE_BITS, jnp.int32)

    def bs_step(_, carry):
        lo, hi = carry
        mid = lo + ((hi - lo) >> 1)
        midf = lax.bitcast_convert_type(mid, jnp.float32)
        cnt = jnp.sum((dsel <= midf).astype(jnp.float32), axis=1, keepdims=True)
        take_lo = cnt >= jnp.float32(_K + 1)
        lo = jnp.where(take_lo, lo, mid + 1)
        hi = jnp.where(take_lo, mid, hi)
        return lo, hi

    lo, hi = lax.fori_loop(0, 31, bs_step, (lo, hi))
    thr = lax.bitcast_convert_type(lo, jnp.float32)

    below = dsel < thr
    pme = pos_mask & below
    nme = neg_mask & below

    cnt_p = jnp.sum(pme.astype(jnp.float32), axis=1, keepdims=True)
    minpos = jnp.min(jnp.where(pos_mask, dist, inf), axis=1, keepdims=True)

    e_a = jnp.exp(_ALPHA * (1.0 - dist))
    pos_logit = jnp.sum(jnp.where(pme, e_a, 0.0), axis=1, keepdims=True)
    neg_logit = jnp.sum(jnp.where(nme, e_a, 0.0), axis=1, keepdims=True)
    pos_beta = jnp.sum(jnp.where(pme, jnp.exp(_BETA * (1.0 - dist)), 0.0),
                       axis=1, keepdims=True)

    has_pos = cnt_p > 0.0
    pos_logit = jnp.where(has_pos, pos_logit, jnp.exp(_ALPHA * (1.0 - minpos)))
    pos_beta = jnp.where(has_pos, pos_beta, jnp.exp(_BETA * (1.0 - minpos)))

    a_lr = 1.0 - pos_logit / (pos_logit + neg_logit)
    pos_loss = -(_ALPHA / _BETA) * jnp.log(pos_beta)
    neg_loss = jnp.log(neg_logit)
    loss = jnp.sum(a_lr * (pos_loss + neg_loss)) / jnp.float32(n)

    posf = pos_mask.astype(jnp.float32)
    negf = neg_mask.astype(jnp.float32)
    pos_sum = jnp.sum(dist * posf)
    neg_sum = jnp.sum(dist * negf)
    pos_cnt = jnp.sum(posf)
    neg_cnt = jnp.sum(negf)

    loss_ref[0, 0] = loss
    posd_ref[0, 0] = pos_sum / pos_cnt
    negd_ref[0, 0] = neg_sum / neg_cnt


@jax.jit
def _nca(inputs, targets):
    n = inputs.shape[0]
    xt = inputs.T
    tcol = targets.reshape(n, 1)
    trow = targets.reshape(1, n)
    scal = jax.ShapeDtypeStruct((1, 1), jnp.float32)
    smem = pl.BlockSpec(memory_space=pltpu.SMEM)
    loss, pos_d, neg_d = pl.pallas_call(
        _nca_body,
        out_shape=(scal, scal, scal),
        out_specs=(smem, smem, smem),
    )(inputs, xt, tcol, trow)
    return loss[0, 0], pos_d[0, 0], neg_d[0, 0]


def kernel(inputs, targets):
    loss, pos_d, neg_d = _nca(inputs, targets)
    return (loss, 0.0, pos_d, neg_d)
